# baseline (speedup 1.0000x reference)
import functools

import jax
import jax.numpy as jnp
from jax import lax
from jax.experimental import pallas as pl
from jax.experimental.pallas import tpu as pltpu
from jax.experimental.pallas import tpu_sc as plsc

_BATCH = 4
_SEQ = 8192
_D = 768
_NC = 2
_NS = 16
_NW = _NC * _NS
_ROWS_PER_W = _SEQ // _NW
_CHUNK = 64
_NCHUNK = _ROWS_PER_W // _CHUNK


def _sc_body(pe_hbm, out_hbm, buf0, buf1, in_sem, out_sem):
    wid = lax.axis_index("s") * _NC + lax.axis_index("c")
    base = wid * _ROWS_PER_W
    bufs = (buf0, buf1)

    def in_copy(k, buf):
        return pltpu.make_async_copy(
            pe_hbm.at[pl.ds(base + k * _CHUNK, _CHUNK), :], buf, in_sem
        )

    in_copy(0, bufs[0]).start()
    for k in range(_NCHUNK):
        buf = bufs[k % 2]
        in_copy(k, buf).wait()
        if k + 1 < _NCHUNK:
            in_copy(k + 1, bufs[(k + 1) % 2]).start()
        outs = [
            pltpu.make_async_copy(
                buf,
                out_hbm.at[b, pl.ds(base + k * _CHUNK, _CHUNK), :],
                out_sem,
            )
            for b in range(_BATCH)
        ]
        for cp in outs:
            cp.start()
        for cp in outs:
            cp.wait()


def kernel(input_ids, pos_embed):
    batch, seq_len = input_ids.shape
    d = pos_embed.shape[1]
    mesh = plsc.VectorSubcoreMesh(core_axis_name="c", subcore_axis_name="s")
    sc_call = pl.kernel(
        _sc_body,
        out_type=jax.ShapeDtypeStruct((batch, seq_len, d), jnp.float32),
        mesh=mesh,
        scratch_types=[
            pltpu.VMEM((_CHUNK, _D), jnp.float32),
            pltpu.VMEM((_CHUNK, _D), jnp.float32),
            pltpu.SemaphoreType.DMA,
            pltpu.SemaphoreType.DMA,
        ],
    )
    return sc_call(pos_embed[:seq_len])


# speedup vs baseline: 1.5280x; 1.5280x over previous
import jax
import jax.numpy as jnp
from jax.experimental import pallas as pl
from jax.experimental.pallas import tpu as pltpu

_BATCH = 4
_BS = 1024


def _body(pe_ref, out_ref):
    out_ref[...] = jnp.broadcast_to(pe_ref[...][None, :, :], out_ref.shape)


def kernel(input_ids, pos_embed):
    batch, seq_len = input_ids.shape
    d = pos_embed.shape[1]
    grid = seq_len // _BS
    return pl.pallas_call(
        _body,
        grid=(grid,),
        in_specs=[pl.BlockSpec((_BS, d), lambda i: (i, 0))],
        out_specs=pl.BlockSpec((batch, _BS, d), lambda i: (0, i, 0)),
        out_shape=jax.ShapeDtypeStruct((batch, seq_len, d), jnp.float32),
        compiler_params=pltpu.CompilerParams(
            dimension_semantics=("arbitrary",),
        ),
    )(pos_embed[:seq_len])
